# Initial kernel scaffold; baseline (speedup 1.0000x reference)
import jax
import jax.numpy as jnp
from jax.experimental import pallas as pl


def kernel(positions, table):
    raise NotImplementedError("write your pallas kernel here")


# speedup vs baseline: 2.2433x; 2.2433x over previous
import functools

import jax
import jax.numpy as jnp
from jax import lax
from jax.experimental import pallas as pl
from jax.experimental.pallas import tpu as pltpu
from jax.experimental.pallas import tpu_sc as plsc

NC = 2
NS = 16
NW = NC * NS
CHUNK = 32
NBUF = 2


@functools.lru_cache(maxsize=None)
def _build(n_chunks: int, d_model: int):
    b_per_w = n_chunks * CHUNK
    total = NW * b_per_w

    mesh = plsc.VectorSubcoreMesh(
        core_axis_name="c", subcore_axis_name="s", num_cores=NC, num_subcores=NS
    )

    @functools.partial(
        pl.kernel,
        out_type=jax.ShapeDtypeStruct((total, d_model), jnp.float32),
        mesh=mesh,
        scratch_types=[
            pltpu.VMEM((n_chunks, CHUNK), jnp.int32),
            pltpu.VMEM((NBUF, CHUNK, d_model), jnp.float32),
            pltpu.SemaphoreType.DMA,
            pltpu.SemaphoreType.DMA,
            pltpu.SemaphoreType.DMA,
            pltpu.SemaphoreType.DMA,
        ],
    )
    def gather_kernel(table_hbm, idx_hbm, out_hbm, idx_v, rows_v, sg0, sg1, sw0, sw1):
        sem_g = [sg0, sg1]
        sem_w = [sw0, sw1]
        wid = lax.axis_index("s") * NC + lax.axis_index("c")
        base = wid * b_per_w

        pltpu.sync_copy(idx_hbm.at[wid], idx_v)

        for b in range(NBUF):
            pltpu.async_copy(table_hbm.at[idx_v.at[b]], rows_v.at[b], sem_g[b])

        @pl.loop(0, n_chunks, step=NBUF)
        def _(g):
            writes = []
            for b in range(NBUF):
                i = g + b
                pltpu.make_async_copy(
                    table_hbm.at[pl.ds(0, CHUNK)], rows_v.at[b], sem_g[b]
                ).wait()
                writes.append(
                    pltpu.async_copy(
                        rows_v.at[b],
                        out_hbm.at[pl.ds(base + i * CHUNK, CHUNK)],
                        sem_w[b],
                    )
                )
            for b in range(NBUF):
                i = g + b
                writes[b].wait()
                nxt = i + NBUF

                @pl.when(nxt < n_chunks)
                def _():
                    pltpu.async_copy(table_hbm.at[idx_v.at[nxt]], rows_v.at[b], sem_g[b])

    return gather_kernel


def kernel(positions, table):
    batch, seq = positions.shape
    d_model = table.shape[1]
    total = batch * seq
    assert total % (NW * CHUNK) == 0
    n_chunks = total // (NW * CHUNK)
    idx = positions.astype(jnp.int32).reshape(NW, n_chunks, CHUNK)
    out = _build(n_chunks, d_model)(table.astype(jnp.float32), idx)
    return out.reshape(batch, seq, d_model)


# speedup vs baseline: 2.2989x; 1.0248x over previous
import functools

import jax
import jax.numpy as jnp
from jax import lax
from jax.experimental import pallas as pl
from jax.experimental.pallas import tpu as pltpu
from jax.experimental.pallas import tpu_sc as plsc

NC = 2
NS = 16
NW = NC * NS
CHUNK = 16
NBUF = 4


@functools.lru_cache(maxsize=None)
def _build(n_chunks: int, d_model: int):
    b_per_w = n_chunks * CHUNK
    total = NW * b_per_w

    mesh = plsc.VectorSubcoreMesh(
        core_axis_name="c", subcore_axis_name="s", num_cores=NC, num_subcores=NS
    )

    @functools.partial(
        pl.kernel,
        out_type=jax.ShapeDtypeStruct((total, d_model), jnp.float32),
        mesh=mesh,
        scratch_types=[
            pltpu.VMEM((n_chunks, CHUNK), jnp.int32),
            pltpu.VMEM((NBUF, CHUNK, d_model), jnp.float32),
        ]
        + [pltpu.SemaphoreType.DMA] * (2 * NBUF),
    )
    def gather_kernel(table_hbm, idx_hbm, out_hbm, idx_v, rows_v, *sems):
        sem_g = list(sems[:NBUF])
        sem_w = list(sems[NBUF:])
        wid = lax.axis_index("s") * NC + lax.axis_index("c")
        base = wid * b_per_w

        pltpu.sync_copy(idx_hbm.at[wid], idx_v)

        for b in range(NBUF):
            pltpu.async_copy(table_hbm.at[idx_v.at[b]], rows_v.at[b], sem_g[b])

        @pl.loop(0, n_chunks, step=NBUF)
        def _(g):
            writes = []
            for b in range(NBUF):
                i = g + b
                pltpu.make_async_copy(
                    table_hbm.at[pl.ds(0, CHUNK)], rows_v.at[b], sem_g[b]
                ).wait()
                writes.append(
                    pltpu.async_copy(
                        rows_v.at[b],
                        out_hbm.at[pl.ds(base + i * CHUNK, CHUNK)],
                        sem_w[b],
                    )
                )
            for b in range(NBUF):
                i = g + b
                writes[b].wait()
                nxt = i + NBUF

                @pl.when(nxt < n_chunks)
                def _():
                    pltpu.async_copy(table_hbm.at[idx_v.at[nxt]], rows_v.at[b], sem_g[b])

    return gather_kernel


def kernel(positions, table):
    batch, seq = positions.shape
    d_model = table.shape[1]
    total = batch * seq
    assert total % (NW * CHUNK) == 0
    n_chunks = total // (NW * CHUNK)
    idx = positions.astype(jnp.int32).reshape(NW, n_chunks, CHUNK)
    out = _build(n_chunks, d_model)(table.astype(jnp.float32), idx)
    return out.reshape(batch, seq, d_model)


# speedup vs baseline: 2.3216x; 1.0099x over previous
import functools

import jax
import jax.numpy as jnp
from jax import lax
from jax.experimental import pallas as pl
from jax.experimental.pallas import tpu as pltpu
from jax.experimental.pallas import tpu_sc as plsc

NC = 2
NS = 16
NW = NC * NS
CHUNK = 8
NBUF = 8


@functools.lru_cache(maxsize=None)
def _build(n_chunks: int, d_model: int):
    b_per_w = n_chunks * CHUNK
    total = NW * b_per_w

    mesh = plsc.VectorSubcoreMesh(
        core_axis_name="c", subcore_axis_name="s", num_cores=NC, num_subcores=NS
    )

    @functools.partial(
        pl.kernel,
        out_type=jax.ShapeDtypeStruct((total, d_model), jnp.float32),
        mesh=mesh,
        scratch_types=[
            pltpu.VMEM((n_chunks, CHUNK), jnp.int32),
            pltpu.VMEM((NBUF, CHUNK, d_model), jnp.float32),
        ]
        + [pltpu.SemaphoreType.DMA] * (2 * NBUF),
    )
    def gather_kernel(table_hbm, idx_hbm, out_hbm, idx_v, rows_v, *sems):
        sem_g = list(sems[:NBUF])
        sem_w = list(sems[NBUF:])
        wid = lax.axis_index("s") * NC + lax.axis_index("c")
        base = wid * b_per_w

        pltpu.sync_copy(idx_hbm.at[wid], idx_v)

        for b in range(NBUF):
            pltpu.async_copy(table_hbm.at[idx_v.at[b]], rows_v.at[b], sem_g[b])

        @pl.loop(0, n_chunks, step=NBUF)
        def _(g):
            writes = []
            for b in range(NBUF):
                i = g + b
                pltpu.make_async_copy(
                    table_hbm.at[pl.ds(0, CHUNK)], rows_v.at[b], sem_g[b]
                ).wait()
                writes.append(
                    pltpu.async_copy(
                        rows_v.at[b],
                        out_hbm.at[pl.ds(base + i * CHUNK, CHUNK)],
                        sem_w[b],
                    )
                )
            for b in range(NBUF):
                i = g + b
                writes[b].wait()
                nxt = i + NBUF

                @pl.when(nxt < n_chunks)
                def _():
                    pltpu.async_copy(table_hbm.at[idx_v.at[nxt]], rows_v.at[b], sem_g[b])

    return gather_kernel


def kernel(positions, table):
    batch, seq = positions.shape
    d_model = table.shape[1]
    total = batch * seq
    assert total % (NW * CHUNK) == 0
    n_chunks = total // (NW * CHUNK)
    idx = positions.astype(jnp.int32).reshape(NW, n_chunks, CHUNK)
    out = _build(n_chunks, d_model)(table.astype(jnp.float32), idx)
    return out.reshape(batch, seq, d_model)
